# Initial kernel scaffold; baseline (speedup 1.0000x reference)
import functools

import jax
import jax.numpy as jnp
from jax import lax
from jax.experimental import pallas as pl
from jax.experimental.pallas import tpu as pltpu
from jax.experimental.pallas import tpu_sc as plsc

VOCAB = 69
EMB_D = 32
OUT_D = 68
BATCH = 16384
SEQ = 20
T = BATCH * SEQ

NC, NS = 2, 16
NW = NC * NS
B_PER_W = T // NW
CHUNK = 128
N_CHUNKS = B_PER_W // CHUNK


def _fuse_body(emb_ref, w_ref, b_ref, out_ref):
    out_ref[...] = (
        lax.dot_general(
            emb_ref[...], w_ref[...],
            dimension_numbers=(((1,), (1,)), ((), ())),
            preferred_element_type=jnp.float32,
        )
        + b_ref[...]
    )


def _fuse_table(emb_table, W, b):
    return pl.pallas_call(
        _fuse_body,
        out_shape=jax.ShapeDtypeStruct((VOCAB, OUT_D), jnp.float32),
    )(emb_table, W, b.reshape(1, OUT_D))


def _sc_body(chars_hbm, emb_hbm, fused_hbm, emb_out, preds_out,
             idx_v, emb_buf, preds_buf, sem1, sem2):
    wid = lax.axis_index("s") * NC + lax.axis_index("c")
    pltpu.sync_copy(chars_hbm.at[wid], idx_v)

    def chunk(j, carry):
        idx_row = idx_v.at[j]
        c1 = pltpu.async_copy(emb_hbm.at[idx_row], emb_buf, sem1)
        c2 = pltpu.async_copy(fused_hbm.at[idx_row], preds_buf, sem2)
        c1.wait()
        c2.wait()
        base = wid * B_PER_W + j * CHUNK
        pltpu.sync_copy(emb_buf, emb_out.at[pl.ds(base, CHUNK)])
        pltpu.sync_copy(preds_buf, preds_out.at[pl.ds(base, CHUNK)])
        return carry

    lax.fori_loop(0, N_CHUNKS, chunk, 0)


def _sc_gather(chars3, emb_table, fused):
    mesh = plsc.VectorSubcoreMesh(core_axis_name="c", subcore_axis_name="s")
    f = pl.kernel(
        _sc_body,
        out_type=(
            jax.ShapeDtypeStruct((T, EMB_D), jnp.float32),
            jax.ShapeDtypeStruct((T, OUT_D), jnp.float32),
        ),
        mesh=mesh,
        scratch_types=(
            pltpu.VMEM((N_CHUNKS, CHUNK), jnp.int32),
            pltpu.VMEM((CHUNK, EMB_D), jnp.float32),
            pltpu.VMEM((CHUNK, OUT_D), jnp.float32),
            pltpu.SemaphoreType.DMA,
            pltpu.SemaphoreType.DMA,
        ),
    )
    return f(chars3, emb_table, fused)


def kernel(chars, emb_table, W, b):
    fused = _fuse_table(emb_table, W, b)
    chars3 = chars.reshape(NW, N_CHUNKS, CHUNK)
    emb_flat, preds_flat = _sc_gather(chars3, emb_table, fused)
    preds = preds_flat.reshape(BATCH, SEQ, OUT_D)
    emb = emb_flat.reshape(BATCH, SEQ, EMB_D)
    return (preds, emb)


# speedup vs baseline: 2.0318x; 2.0318x over previous
import jax
import jax.numpy as jnp
from jax import lax
from jax.experimental import pallas as pl
from jax.experimental.pallas import tpu as pltpu
from jax.experimental.pallas import tpu_sc as plsc

VOCAB = 69
EMB_D = 32
OUT_D = 68
VPAD = 128
BATCH = 16384
SEQ = 20
T = BATCH * SEQ

NC, NS = 2, 16
NW = NC * NS
B_PER_W = T // NW
CHUNK = 128
N_CHUNKS = B_PER_W // CHUNK

TB = 2048
NB = T // TB


def _sc_body(chars_hbm, emb_hbm, emb_out, idx_v, emb_buf, sem1):
    wid = lax.axis_index("s") * NC + lax.axis_index("c")
    pltpu.sync_copy(chars_hbm.at[wid], idx_v)

    def chunk(j, carry):
        idx_row = idx_v.at[j]
        pltpu.async_copy(emb_hbm.at[idx_row], emb_buf, sem1).wait()
        base = wid * B_PER_W + j * CHUNK
        pltpu.sync_copy(emb_buf, emb_out.at[pl.ds(base, CHUNK)])
        return carry

    lax.fori_loop(0, N_CHUNKS, chunk, 0)


def _sc_gather(chars3, emb_table):
    mesh = plsc.VectorSubcoreMesh(core_axis_name="c", subcore_axis_name="s")
    f = pl.kernel(
        _sc_body,
        out_type=jax.ShapeDtypeStruct((T, EMB_D), jnp.float32),
        mesh=mesh,
        scratch_types=(
            pltpu.VMEM((N_CHUNKS, CHUNK), jnp.int32),
            pltpu.VMEM((CHUNK, EMB_D), jnp.float32),
            pltpu.SemaphoreType.DMA,
        ),
        compiler_params=pltpu.CompilerParams(use_tc_tiling_on_sc=False),
    )
    return f(chars3, emb_table)


def _tc_preds_body(chars_ref, emb_ref, w_ref, b_ref, out_ref, fused_ref):
    @pl.when(pl.program_id(0) == 0)
    def _():
        fused = lax.dot_general(
            emb_ref[...], w_ref[...],
            dimension_numbers=(((1,), (1,)), ((), ())),
            preferred_element_type=jnp.float32,
        ) + b_ref[...]
        fused_ref[...] = fused.astype(jnp.bfloat16)

    cb = chars_ref[0]
    iota = lax.broadcasted_iota(jnp.int32, (TB, VPAD), 1)
    onehot = (iota == cb).astype(jnp.bfloat16)
    res = lax.dot_general(
        onehot, fused_ref[...],
        dimension_numbers=(((1,), (0,)), ((), ())),
        preferred_element_type=jnp.float32,
    )
    out_ref[...] = lax.slice(res, (0, 0), (TB, OUT_D))


def _tc_preds(chars2, emb_pad, w_pad, b_pad):
    return pl.pallas_call(
        _tc_preds_body,
        grid=(NB,),
        in_specs=[
            pl.BlockSpec((1, TB, 1), lambda i: (i, 0, 0)),
            pl.BlockSpec((VPAD, EMB_D), lambda i: (0, 0)),
            pl.BlockSpec((VPAD, EMB_D), lambda i: (0, 0)),
            pl.BlockSpec((1, VPAD), lambda i: (0, 0)),
        ],
        out_specs=pl.BlockSpec((TB, OUT_D), lambda i: (i, 0)),
        out_shape=jax.ShapeDtypeStruct((T, OUT_D), jnp.float32),
        scratch_shapes=[pltpu.VMEM((VPAD, VPAD), jnp.bfloat16)],
    )(chars2, emb_pad, w_pad, b_pad)


def kernel(chars, emb_table, W, b):
    chars3 = chars.reshape(NW, N_CHUNKS, CHUNK)
    emb_flat = _sc_gather(chars3, emb_table)

    emb_pad = jnp.pad(emb_table, ((0, VPAD - VOCAB), (0, 0)))
    w_pad = jnp.pad(W, ((0, VPAD - OUT_D), (0, 0)))
    b_pad = jnp.pad(b, (0, VPAD - OUT_D)).reshape(1, VPAD)
    chars2 = chars.reshape(NB, TB, 1)
    preds_flat = _tc_preds(chars2, emb_pad, w_pad, b_pad)

    preds = preds_flat.reshape(BATCH, SEQ, OUT_D)
    emb = emb_flat.reshape(BATCH, SEQ, EMB_D)
    return (preds, emb)


# speedup vs baseline: 3.8796x; 1.9094x over previous
import jax
import jax.numpy as jnp
from jax import lax
from jax.experimental import pallas as pl
from jax.experimental.pallas import tpu as pltpu
from jax.experimental.pallas import tpu_sc as plsc

VOCAB = 69
VPAD = 72
EMB_D = 32
OUT_D = 68
BATCH = 16384
SEQ = 20
T = BATCH * SEQ

NC, NS = 2, 16
NW = NC * NS
B_PER_W = T // NW
CHUNK = 128
GC = 8
GROUP = GC * CHUNK
NGRP = B_PER_W // GROUP

TBB = 4096
NBB = BATCH // TBB


def _sc_body(chars_hbm, emb_hbm, emb_out,
             idx_v, buf_a, buf_b, gsem_a, gsem_b, wsem_a, wsem_b):
    w = lax.axis_index("s") * NC + lax.axis_index("c")
    pltpu.sync_copy(chars_hbm.at[w], idx_v)

    wb = [None, None]
    for g in range(NGRP):
        s = g % 2
        buf = buf_a if s == 0 else buf_b
        gsem = gsem_a if s == 0 else gsem_b
        wsem = wsem_a if s == 0 else wsem_b
        if wb[s] is not None:
            wb[s].wait()
        gh = []
        for c in range(GC):
            j = g * GC + c
            gh.append(pltpu.async_copy(
                emb_hbm.at[idx_v.at[j]],
                buf.at[pl.ds(c * CHUNK, CHUNK)], gsem))
        for h in gh:
            h.wait()
        wb[s] = pltpu.async_copy(
            buf, emb_out.at[pl.ds(w * B_PER_W + g * GROUP, GROUP)], wsem)
    wb[0].wait()
    wb[1].wait()


def _sc_emb(chars3, emb_table):
    mesh = plsc.VectorSubcoreMesh(core_axis_name="c", subcore_axis_name="s")
    f = pl.kernel(
        _sc_body,
        out_type=jax.ShapeDtypeStruct((T, EMB_D), jnp.float32),
        mesh=mesh,
        scratch_types=(
            pltpu.VMEM((B_PER_W // CHUNK, CHUNK), jnp.int32),
            pltpu.VMEM((GROUP, EMB_D), jnp.float32),
            pltpu.VMEM((GROUP, EMB_D), jnp.float32),
            pltpu.SemaphoreType.DMA,
            pltpu.SemaphoreType.DMA,
            pltpu.SemaphoreType.DMA,
            pltpu.SemaphoreType.DMA,
        ),
        compiler_params=pltpu.CompilerParams(use_tc_tiling_on_sc=False),
    )
    return f(chars3, emb_table)


def _tc_preds_body(chars_ref, wt_ref, embtt_ref, bcol_ref, out_ref, fused_ref):
    @pl.when((pl.program_id(0) == 0) & (pl.program_id(1) == 0))
    def _():
        ft = lax.dot_general(
            wt_ref[...], embtt_ref[...],
            dimension_numbers=(((0,), (0,)), ((), ())),
            preferred_element_type=jnp.float32,
        ) + bcol_ref[...]
        fused_ref[...] = jnp.zeros((VPAD, VPAD), jnp.bfloat16)
        fused_ref[0:OUT_D, 0:VOCAB] = ft.astype(jnp.bfloat16)

    s = pl.program_id(1)
    cb = chars_ref[pl.ds(s, 1), :]
    iota = lax.broadcasted_iota(jnp.int32, (VPAD, TBB), 0)
    onehot = (iota == cb).astype(jnp.bfloat16)
    res = lax.dot_general(
        fused_ref[...], onehot,
        dimension_numbers=(((1,), (0,)), ((), ())),
        preferred_element_type=jnp.float32,
    )
    out_ref[...] = lax.slice(res, (0, 0), (OUT_D, TBB)).reshape(1, OUT_D, TBB)


def _tc_preds(chars_t, wt, embtt, bcol):
    return pl.pallas_call(
        _tc_preds_body,
        grid=(NBB, SEQ),
        in_specs=[
            pl.BlockSpec((SEQ, TBB), lambda t, s: (0, t)),
            pl.BlockSpec((EMB_D, OUT_D), lambda t, s: (0, 0)),
            pl.BlockSpec((EMB_D, VOCAB), lambda t, s: (0, 0)),
            pl.BlockSpec((OUT_D, 1), lambda t, s: (0, 0)),
        ],
        out_specs=pl.BlockSpec((1, OUT_D, TBB), lambda t, s: (s, 0, t)),
        out_shape=jax.ShapeDtypeStruct((SEQ, OUT_D, BATCH), jnp.float32),
        scratch_shapes=[pltpu.VMEM((VPAD, VPAD), jnp.bfloat16)],
    )(chars_t, wt, embtt, bcol)


def kernel(chars, emb_table, W, b):
    chars3 = chars.reshape(NW, B_PER_W // CHUNK, CHUNK)
    emb_flat = _sc_emb(chars3, emb_table)

    chars_t = chars.T
    embtt = emb_table.T
    wt = W.T
    bcol = b.reshape(OUT_D, 1)
    preds_t = _tc_preds(chars_t, wt, embtt, bcol)

    preds = preds_t.transpose(2, 0, 1)
    emb = emb_flat.reshape(BATCH, SEQ, EMB_D)
    return (preds, emb)


# speedup vs baseline: 4.1160x; 1.0610x over previous
import jax
import jax.numpy as jnp
from jax import lax
from jax.experimental import pallas as pl
from jax.experimental.pallas import tpu as pltpu
from jax.experimental.pallas import tpu_sc as plsc

VOCAB = 69
VPAD = 72
EMB_D = 32
OUT_D = 68
BATCH = 16384
SEQ = 20
T = BATCH * SEQ

NC, NS = 2, 16
NW = NC * NS
B_PER_W = T // NW
CHUNK = 128
GC = 8
GROUP = GC * CHUNK
NGRP = B_PER_W // GROUP

TBB = 4096
NBB = BATCH // TBB


def _sc_body(chars_hbm, emb_hbm, emb_out,
             idx_v, buf_a, buf_b, gsem_a, gsem_b, wsem_a, wsem_b):
    w = lax.axis_index("s") * NC + lax.axis_index("c")
    pltpu.sync_copy(chars_hbm.at[w], idx_v)

    wb = [None, None]
    for g in range(NGRP):
        s = g % 2
        buf = buf_a if s == 0 else buf_b
        gsem = gsem_a if s == 0 else gsem_b
        wsem = wsem_a if s == 0 else wsem_b
        if wb[s] is not None:
            wb[s].wait()
        gh = []
        for c in range(GC):
            j = g * GC + c
            gh.append(pltpu.async_copy(
                emb_hbm.at[idx_v.at[j]],
                buf.at[pl.ds(c * CHUNK, CHUNK)], gsem))
        for h in gh:
            h.wait()
        wb[s] = pltpu.async_copy(
            buf, emb_out.at[pl.ds(w * B_PER_W + g * GROUP, GROUP)], wsem)
    wb[0].wait()
    wb[1].wait()


def _sc_emb(chars3, emb_table):
    mesh = plsc.VectorSubcoreMesh(core_axis_name="c", subcore_axis_name="s")
    f = pl.kernel(
        _sc_body,
        out_type=jax.ShapeDtypeStruct((T, EMB_D), jnp.float32),
        mesh=mesh,
        scratch_types=(
            pltpu.VMEM((B_PER_W // CHUNK, CHUNK), jnp.int32),
            pltpu.VMEM((GROUP, EMB_D), jnp.float32),
            pltpu.VMEM((GROUP, EMB_D), jnp.float32),
            pltpu.SemaphoreType.DMA,
            pltpu.SemaphoreType.DMA,
            pltpu.SemaphoreType.DMA,
            pltpu.SemaphoreType.DMA,
        ),
        compiler_params=pltpu.CompilerParams(use_tc_tiling_on_sc=False),
    )
    return f(chars3, emb_table)


def _tc_preds_body(chars_ref, wt_ref, embtt_ref, bcol_ref, out_ref, fused_ref):
    @pl.when((pl.program_id(0) == 0) & (pl.program_id(1) == 0))
    def _():
        ft = lax.dot_general(
            wt_ref[...], embtt_ref[...],
            dimension_numbers=(((0,), (0,)), ((), ())),
            preferred_element_type=jnp.float32,
        ) + bcol_ref[...]
        fused_ref[...] = jnp.zeros((VPAD, VPAD), jnp.bfloat16)
        fused_ref[0:OUT_D, 0:VOCAB] = ft.astype(jnp.bfloat16)

    s = pl.program_id(1)
    cb = chars_ref[pl.ds(s, 1), :]
    iota = lax.broadcasted_iota(jnp.int32, (VPAD, TBB), 0)
    onehot = (iota == cb).astype(jnp.bfloat16)
    res = lax.dot_general(
        fused_ref[...], onehot,
        dimension_numbers=(((1,), (0,)), ((), ())),
        preferred_element_type=jnp.float32,
    )
    out_ref[...] = lax.slice(res, (0, 0), (OUT_D, TBB)).reshape(1, OUT_D, TBB)


def _tc_preds(chars_t, wt, embtt, bcol):
    return pl.pallas_call(
        _tc_preds_body,
        grid=(NBB, SEQ),
        in_specs=[
            pl.BlockSpec((SEQ, TBB), lambda t, s: (0, t)),
            pl.BlockSpec((EMB_D, OUT_D), lambda t, s: (0, 0)),
            pl.BlockSpec((EMB_D, VOCAB), lambda t, s: (0, 0)),
            pl.BlockSpec((OUT_D, 1), lambda t, s: (0, 0)),
        ],
        out_specs=pl.BlockSpec((1, OUT_D, TBB), lambda t, s: (s, 0, t)),
        out_shape=jax.ShapeDtypeStruct((SEQ, OUT_D, BATCH), jnp.float32),
        scratch_shapes=[pltpu.VMEM((VPAD, VPAD), jnp.bfloat16)],
    )(chars_t, wt, embtt, bcol)


def kernel(chars, emb_table, W, b):
    chars_t = chars.T

    chars3 = chars_t.reshape(NW, B_PER_W // CHUNK, CHUNK)
    emb_p = _sc_emb(chars3, emb_table)

    embtt = emb_table.T
    wt = W.T
    bcol = b.reshape(OUT_D, 1)
    preds_t = _tc_preds(chars_t, wt, embtt, bcol)

    preds = preds_t.transpose(2, 0, 1)
    emb = emb_p.reshape(SEQ, BATCH, EMB_D).transpose(1, 0, 2)
    return (preds, emb)


# speedup vs baseline: 4.1550x; 1.0095x over previous
import jax
import jax.numpy as jnp
from jax import lax
from jax.experimental import pallas as pl
from jax.experimental.pallas import tpu as pltpu
from jax.experimental.pallas import tpu_sc as plsc

VOCAB = 69
VPAD = 72
EMB_D = 32
OUT_D = 68
BATCH = 16384
SEQ = 20
T = BATCH * SEQ

NC, NS = 2, 16
NW = NC * NS
B_PER_W = T // NW
CHUNK = 128
GC = 8
GROUP = GC * CHUNK
NGRP = B_PER_W // GROUP

TBB = 4096
NBB = BATCH // TBB


def _sc_body(chars_hbm, emb_hbm, emb_out,
             idx_v, buf_a, buf_b, gsem_a, gsem_b, wsem_a, wsem_b):
    w = lax.axis_index("s") * NC + lax.axis_index("c")
    pltpu.sync_copy(chars_hbm.at[w], idx_v)

    bufs = (buf_a, buf_b)
    gsems = (gsem_a, gsem_b)
    wsems = (wsem_a, wsem_b)
    wb = [None, None]
    gh = [[], []]

    def fire(g):
        s = g % 2
        if wb[s] is not None:
            wb[s].wait()
        for c in range(GC):
            j = g * GC + c
            gh[s].append(pltpu.async_copy(
                emb_hbm.at[idx_v.at[j]],
                bufs[s].at[pl.ds(c * CHUNK, CHUNK)], gsems[s]))

    def drain_and_writeback(g):
        s = g % 2
        for h in gh[s]:
            h.wait()
        gh[s] = []
        wb[s] = pltpu.async_copy(
            bufs[s], emb_out.at[pl.ds(w * B_PER_W + g * GROUP, GROUP)],
            wsems[s])

    fire(0)
    for g in range(1, NGRP):
        fire(g)
        drain_and_writeback(g - 1)
    drain_and_writeback(NGRP - 1)
    wb[0].wait()
    wb[1].wait()


def _sc_emb(chars3, emb_table):
    mesh = plsc.VectorSubcoreMesh(core_axis_name="c", subcore_axis_name="s")
    f = pl.kernel(
        _sc_body,
        out_type=jax.ShapeDtypeStruct((T, EMB_D), jnp.float32),
        mesh=mesh,
        scratch_types=(
            pltpu.VMEM((B_PER_W // CHUNK, CHUNK), jnp.int32),
            pltpu.VMEM((GROUP, EMB_D), jnp.float32),
            pltpu.VMEM((GROUP, EMB_D), jnp.float32),
            pltpu.SemaphoreType.DMA,
            pltpu.SemaphoreType.DMA,
            pltpu.SemaphoreType.DMA,
            pltpu.SemaphoreType.DMA,
        ),
        compiler_params=pltpu.CompilerParams(use_tc_tiling_on_sc=False),
    )
    return f(chars3, emb_table)


def _tc_preds_body(chars_ref, wt_ref, embtt_ref, bcol_ref, out_ref, fused_ref):
    @pl.when((pl.program_id(0) == 0) & (pl.program_id(1) == 0))
    def _():
        ft = lax.dot_general(
            wt_ref[...], embtt_ref[...],
            dimension_numbers=(((0,), (0,)), ((), ())),
            preferred_element_type=jnp.float32,
        ) + bcol_ref[...]
        fused_ref[...] = jnp.zeros((VPAD, VPAD), jnp.bfloat16)
        fused_ref[0:OUT_D, 0:VOCAB] = ft.astype(jnp.bfloat16)

    s = pl.program_id(1)
    cb = chars_ref[pl.ds(s, 1), :]
    iota = lax.broadcasted_iota(jnp.int32, (VPAD, TBB), 0)
    onehot = (iota == cb).astype(jnp.bfloat16)
    res = lax.dot_general(
        fused_ref[...], onehot,
        dimension_numbers=(((1,), (0,)), ((), ())),
        preferred_element_type=jnp.float32,
    )
    out_ref[...] = lax.slice(res, (0, 0), (OUT_D, TBB)).reshape(1, OUT_D, TBB)


def _tc_preds(chars_t, wt, embtt, bcol):
    return pl.pallas_call(
        _tc_preds_body,
        grid=(NBB, SEQ),
        in_specs=[
            pl.BlockSpec((SEQ, TBB), lambda t, s: (0, t)),
            pl.BlockSpec((EMB_D, OUT_D), lambda t, s: (0, 0)),
            pl.BlockSpec((EMB_D, VOCAB), lambda t, s: (0, 0)),
            pl.BlockSpec((OUT_D, 1), lambda t, s: (0, 0)),
        ],
        out_specs=pl.BlockSpec((1, OUT_D, TBB), lambda t, s: (s, 0, t)),
        out_shape=jax.ShapeDtypeStruct((SEQ, OUT_D, BATCH), jnp.float32),
        scratch_shapes=[pltpu.VMEM((VPAD, VPAD), jnp.bfloat16)],
    )(chars_t, wt, embtt, bcol)


def kernel(chars, emb_table, W, b):
    chars_t = chars.T

    chars3 = chars_t.reshape(NW, B_PER_W // CHUNK, CHUNK)
    emb_p = _sc_emb(chars3, emb_table)

    embtt = emb_table.T
    wt = W.T
    bcol = b.reshape(OUT_D, 1)
    preds_t = _tc_preds(chars_t, wt, embtt, bcol)

    preds = preds_t.transpose(2, 0, 1)
    emb = emb_p.reshape(SEQ, BATCH, EMB_D).transpose(1, 0, 2)
    return (preds, emb)


# speedup vs baseline: 8.2209x; 1.9786x over previous
import jax
import jax.numpy as jnp
from jax import lax
from jax.experimental import pallas as pl
from jax.experimental.pallas import tpu as pltpu
from jax.experimental.pallas import tpu_sc as plsc

VOCAB = 69
VPAD = 72
EMB_D = 32
OUT_D = 68
BATCH = 16384
SEQ = 20
T = BATCH * SEQ

NC, NS = 2, 16
NW = NC * NS
B_PER_W = T // NW
CHUNK = 128
GC = 8
GROUP = GC * CHUNK
NGRP = B_PER_W // GROUP

TBB = 4096
NBB = BATCH // TBB


def _sc_body(chars_hbm, emb_hbm, emb_out,
             idx_v, tab_v, buf_a, buf_b, gsem_a, gsem_b, wsem_a, wsem_b):
    w = lax.axis_index("s") * NC + lax.axis_index("c")
    @pl.when(lax.axis_index("s") == 0)
    def _():
        pltpu.sync_copy(emb_hbm, tab_v)

    plsc.subcore_barrier()
    pltpu.sync_copy(chars_hbm.at[w], idx_v)

    bufs = (buf_a, buf_b)
    gsems = (gsem_a, gsem_b)
    wsems = (wsem_a, wsem_b)
    wb = [None, None]
    gh = [[], []]

    def fire(g):
        s = g % 2
        if wb[s] is not None:
            wb[s].wait()
        for c in range(GC):
            j = g * GC + c
            gh[s].append(pltpu.async_copy(
                tab_v.at[idx_v.at[j]],
                bufs[s].at[pl.ds(c * CHUNK, CHUNK)], gsems[s]))

    def drain_and_writeback(g):
        s = g % 2
        for h in gh[s]:
            h.wait()
        gh[s] = []
        wb[s] = pltpu.async_copy(
            bufs[s], emb_out.at[pl.ds(w * B_PER_W + g * GROUP, GROUP)],
            wsems[s])

    fire(0)
    for g in range(1, NGRP):
        fire(g)
        drain_and_writeback(g - 1)
    drain_and_writeback(NGRP - 1)
    wb[0].wait()
    wb[1].wait()


def _sc_emb(chars3, emb_table):
    mesh = plsc.VectorSubcoreMesh(core_axis_name="c", subcore_axis_name="s")
    f = pl.kernel(
        _sc_body,
        out_type=jax.ShapeDtypeStruct((T, EMB_D), jnp.float32),
        mesh=mesh,
        scratch_types=(
            pltpu.VMEM((B_PER_W // CHUNK, CHUNK), jnp.int32),
            pltpu.VMEM_SHARED((VOCAB, EMB_D), jnp.float32),
            pltpu.VMEM((GROUP, EMB_D), jnp.float32),
            pltpu.VMEM((GROUP, EMB_D), jnp.float32),
            pltpu.SemaphoreType.DMA,
            pltpu.SemaphoreType.DMA,
            pltpu.SemaphoreType.DMA,
            pltpu.SemaphoreType.DMA,
        ),
        compiler_params=pltpu.CompilerParams(use_tc_tiling_on_sc=False),
    )
    return f(chars3, emb_table)


def _tc_preds_body(chars_ref, wt_ref, embtt_ref, bcol_ref, out_ref, fused_ref):
    @pl.when((pl.program_id(0) == 0) & (pl.program_id(1) == 0))
    def _():
        ft = lax.dot_general(
            wt_ref[...], embtt_ref[...],
            dimension_numbers=(((0,), (0,)), ((), ())),
            preferred_element_type=jnp.float32,
        ) + bcol_ref[...]
        fused_ref[...] = jnp.zeros((VPAD, VPAD), jnp.bfloat16)
        fused_ref[0:OUT_D, 0:VOCAB] = ft.astype(jnp.bfloat16)

    s = pl.program_id(1)
    cb = chars_ref[pl.ds(s, 1), :]
    iota = lax.broadcasted_iota(jnp.int32, (VPAD, TBB), 0)
    onehot = (iota == cb).astype(jnp.bfloat16)
    res = lax.dot_general(
        fused_ref[...], onehot,
        dimension_numbers=(((1,), (0,)), ((), ())),
        preferred_element_type=jnp.float32,
    )
    out_ref[...] = lax.slice(res, (0, 0), (OUT_D, TBB)).reshape(1, OUT_D, TBB)


def _tc_preds(chars_t, wt, embtt, bcol):
    return pl.pallas_call(
        _tc_preds_body,
        grid=(NBB, SEQ),
        in_specs=[
            pl.BlockSpec((SEQ, TBB), lambda t, s: (0, t)),
            pl.BlockSpec((EMB_D, OUT_D), lambda t, s: (0, 0)),
            pl.BlockSpec((EMB_D, VOCAB), lambda t, s: (0, 0)),
            pl.BlockSpec((OUT_D, 1), lambda t, s: (0, 0)),
        ],
        out_specs=pl.BlockSpec((1, OUT_D, TBB), lambda t, s: (s, 0, t)),
        out_shape=jax.ShapeDtypeStruct((SEQ, OUT_D, BATCH), jnp.float32),
        scratch_shapes=[pltpu.VMEM((VPAD, VPAD), jnp.bfloat16)],
    )(chars_t, wt, embtt, bcol)


def kernel(chars, emb_table, W, b):
    chars_t = chars.T

    chars3 = chars_t.reshape(NW, B_PER_W // CHUNK, CHUNK)
    emb_p = _sc_emb(chars3, emb_table)

    embtt = emb_table.T
    wt = W.T
    bcol = b.reshape(OUT_D, 1)
    preds_t = _tc_preds(chars_t, wt, embtt, bcol)

    preds = preds_t.transpose(2, 0, 1)
    emb = emb_p.reshape(SEQ, BATCH, EMB_D).transpose(1, 0, 2)
    return (preds, emb)
